# baseline (speedup 1.0000x reference)
import jax
import jax.numpy as jnp
from jax import lax
from jax.experimental import pallas as pl

_K, _D = 1024, 32


def _mm_body(x_ref, w_ref, b_ref, o_ref):
    xb = x_ref[0]
    ib, wd, kd = xb.shape
    x2 = xb.reshape(ib * wd, kd)
    acc = jnp.dot(x2, w_ref[...], preferred_element_type=jnp.float32)
    acc = jnp.maximum(acc + b_ref[...], 0.0)
    o_ref[0] = acc.reshape(ib, wd, acc.shape[-1]).astype(o_ref.dtype)


def _conv_mm(x, w, b, ib, out_dtype=jnp.float32):
    n, hh, ww, kd = x.shape
    co = w.shape[1]
    return pl.pallas_call(
        _mm_body,
        grid=(n, hh // ib),
        in_specs=[
            pl.BlockSpec((1, ib, ww, kd), lambda ni, i: (ni, i, 0, 0)),
            pl.BlockSpec((kd, co), lambda ni, i: (0, 0)),
            pl.BlockSpec((1, co), lambda ni, i: (0, 0)),
        ],
        out_specs=pl.BlockSpec((1, ib, ww, co), lambda ni, i: (ni, i, 0, 0)),
        out_shape=jax.ShapeDtypeStruct((n, hh, ww, co), out_dtype),
    )(x, w, b)


_VQB = 256


def _vq_body(z_ref, s_ref, cb_ref, zq_ref):
    zb = z_ref[...]
    a = jnp.sum(zb * zb, axis=1, keepdims=True)
    d = jnp.abs(a - s_ref[...])
    m = jnp.min(d, axis=1, keepdims=True)
    iota = lax.broadcasted_iota(jnp.int32, (_VQB, _K), 1)
    idx = jnp.min(jnp.where(d == m, iota, _K), axis=1)
    onehot = (lax.broadcasted_iota(jnp.int32, (_VQB, _K), 1)
              == idx[:, None]).astype(jnp.float32)
    zq_ref[...] = jnp.dot(onehot, cb_ref[...], preferred_element_type=jnp.float32)


def _vq(zflat, s, codebook):
    n = zflat.shape[0]
    return pl.pallas_call(
        _vq_body,
        grid=(n // _VQB,),
        in_specs=[
            pl.BlockSpec((_VQB, _D), lambda i: (i, 0)),
            pl.BlockSpec((1, _K), lambda i: (0, 0)),
            pl.BlockSpec((_K, _D), lambda i: (0, 0)),
        ],
        out_specs=pl.BlockSpec((_VQB, _D), lambda i: (i, 0)),
        out_shape=jax.ShapeDtypeStruct((n, _D), jnp.float32),
    )(zflat, s, codebook)


def _s2d(x_cl):
    n, hh, ww, c = x_cl.shape
    return (x_cl.reshape(n, hh // 2, 2, ww // 2, 2, c)
            .transpose(0, 1, 3, 2, 4, 5)
            .reshape(n, hh // 2, ww // 2, 4 * c))


def _win4(xq, hout):
    return jnp.concatenate(
        [xq[:, a:a + hout, b:b + hout, :] for a in (0, 1) for b in (0, 1)],
        axis=-1)


def _win9(xp, hout):
    return jnp.concatenate(
        [xp[:, a:a + hout, b:b + hout, :] for a in (0, 1, 2) for b in (0, 1, 2)],
        axis=-1)


def _enc_weight(w):
    co, ci = w.shape[0], w.shape[1]
    return (w.reshape(co, ci, 2, 2, 2, 2)
            .transpose(2, 4, 3, 5, 1, 0)
            .reshape(16 * ci, co))


_KI = ((0, 2, -1), (-1, 1, 3))


def _dec_weight(w):
    ci, co = w.shape[0], w.shape[1]
    wf = jnp.flip(w, axis=(2, 3)).transpose(1, 0, 2, 3)
    big = jnp.zeros((3, 3, ci, 2, 2, co), w.dtype)
    for r in (0, 1):
        for si in (0, 1, 2):
            ki = _KI[r][si]
            if ki < 0:
                continue
            for c in (0, 1):
                for sj in (0, 1, 2):
                    kj = _KI[c][sj]
                    if kj < 0:
                        continue
                    big = big.at[si, sj, :, r, c, :].set(wf[:, :, ki, kj].T)
    return big.reshape(9 * ci, 4 * co)


def _pad1(x_cl):
    return jnp.pad(x_cl, ((0, 0), (1, 1), (1, 1), (0, 0)))


def kernel(imgs, w1, b1, w2, b2, codebook, wt1, bt1, wt2, bt2):
    n = imgs.shape[0]

    imp = _s2d(_pad1(imgs.transpose(0, 2, 3, 1)))
    x1 = _win4(imp, 192)
    h_cl = _conv_mm(x1, _enc_weight(w1), b1[None, :], 48)

    hq = _s2d(_pad1(h_cl))
    x2 = _win4(hq, 96)
    ze_cl = _conv_mm(x2, _enc_weight(w2), b2[None, :], 24)

    zflat = ze_cl.reshape(-1, _D)
    s = jnp.sum(codebook ** 2, axis=1)
    zq = _vq(zflat, s[None, :], codebook)
    zq_cl = zq.reshape(n, 96, 96, _D)

    xt1 = _win9(_pad1(zq_cl), 96).astype(jnp.bfloat16)
    d4 = _conv_mm(xt1, _dec_weight(wt1).astype(jnp.bfloat16),
                  jnp.tile(bt1, 4)[None, :], 24)
    d_cl = (d4.reshape(n, 96, 96, 2, 2, 16)
            .transpose(0, 1, 3, 2, 4, 5)
            .reshape(n, 192, 192, 16))

    xt2 = _win9(_pad1(d_cl), 192).astype(jnp.bfloat16)
    dec4 = _conv_mm(xt2, _dec_weight(wt2).astype(jnp.bfloat16),
                    jnp.tile(bt2, 4)[None, :], 48)
    decoded = (dec4.reshape(n, 192, 192, 2, 2, 3)
               .transpose(0, 1, 3, 2, 4, 5)
               .reshape(n, 384, 384, 3)
               .transpose(0, 3, 1, 2))

    z_e = ze_cl.transpose(0, 3, 1, 2)
    encoded = zq_cl.transpose(0, 3, 1, 2)
    return (z_e, encoded, decoded)


# speedup vs baseline: 1.1337x; 1.1337x over previous
import jax
import jax.numpy as jnp
from jax import lax
from jax.experimental import pallas as pl

_K, _D = 1024, 32


def _mm_multi_body(*refs):
    o_ref = refs[-1]
    b_ref = refs[-2]
    w_ref = refs[-3]
    xs = refs[:-3]
    ib, wd, kd = xs[0][0].shape
    acc = None
    for p, x_ref in enumerate(xs):
        x2 = x_ref[0].reshape(ib * wd, kd)
        t = jnp.dot(x2, w_ref[p], preferred_element_type=jnp.float32)
        acc = t if acc is None else acc + t
    acc = jnp.maximum(acc + b_ref[...], 0.0)
    o_ref[0] = acc.reshape(ib, wd, acc.shape[-1]).astype(o_ref.dtype)


def _conv_mm(xs, w, b, ib, out_dtype=jnp.float32):
    n, hh, ww, kd = xs[0].shape
    npc = len(xs)
    co = w.shape[-1]
    return pl.pallas_call(
        _mm_multi_body,
        grid=(n, hh // ib),
        in_specs=(
            [pl.BlockSpec((1, ib, ww, kd), lambda ni, i: (ni, i, 0, 0))] * npc
            + [pl.BlockSpec((npc, kd, co), lambda ni, i: (0, 0, 0)),
               pl.BlockSpec((1, co), lambda ni, i: (0, 0))]),
        out_specs=pl.BlockSpec((1, ib, ww, co), lambda ni, i: (ni, i, 0, 0)),
        out_shape=jax.ShapeDtypeStruct((n, hh, ww, co), out_dtype),
    )(*xs, w, b)


_VQB = 256


def _vq_body(z_ref, s_ref, cb_ref, zq_ref):
    zb = z_ref[...]
    a = jnp.sum(zb * zb, axis=1, keepdims=True)
    d = jnp.abs(a - s_ref[...])
    m = jnp.min(d, axis=1, keepdims=True)
    iota = lax.broadcasted_iota(jnp.int32, (_VQB, _K), 1)
    idx = jnp.min(jnp.where(d == m, iota, _K), axis=1)
    onehot = (lax.broadcasted_iota(jnp.int32, (_VQB, _K), 1)
              == idx[:, None]).astype(jnp.float32)
    zq_ref[...] = jnp.dot(onehot, cb_ref[...], preferred_element_type=jnp.float32)


def _vq(zflat, s, codebook):
    n = zflat.shape[0]
    return pl.pallas_call(
        _vq_body,
        grid=(n // _VQB,),
        in_specs=[
            pl.BlockSpec((_VQB, _D), lambda i: (i, 0)),
            pl.BlockSpec((1, _K), lambda i: (0, 0)),
            pl.BlockSpec((_K, _D), lambda i: (0, 0)),
        ],
        out_specs=pl.BlockSpec((_VQB, _D), lambda i: (i, 0)),
        out_shape=jax.ShapeDtypeStruct((n, _D), jnp.float32),
    )(zflat, s, codebook)


def _s2d(x_cl):
    n, hh, ww, c = x_cl.shape
    return (x_cl.reshape(n, hh // 2, 2, ww // 2, 2, c)
            .transpose(0, 1, 3, 2, 4, 5)
            .reshape(n, hh // 2, ww // 2, 4 * c))


def _win4(xq, hout):
    return [xq[:, a:a + hout, b:b + hout, :] for a in (0, 1) for b in (0, 1)]


def _win9(xp, hout):
    return [xp[:, a:a + hout, b:b + hout, :] for a in (0, 1, 2) for b in (0, 1, 2)]


def _enc_weight(w):
    co, ci = w.shape[0], w.shape[1]
    return (w.reshape(co, ci, 2, 2, 2, 2)
            .transpose(2, 4, 3, 5, 1, 0)
            .reshape(4, 4 * ci, co))


_KI = ((0, 2, -1), (-1, 1, 3))


def _dec_weight(w):
    ci, co = w.shape[0], w.shape[1]
    wf = jnp.flip(w, axis=(2, 3)).transpose(1, 0, 2, 3)
    big = jnp.zeros((3, 3, ci, 2, 2, co), w.dtype)
    for r in (0, 1):
        for si in (0, 1, 2):
            ki = _KI[r][si]
            if ki < 0:
                continue
            for c in (0, 1):
                for sj in (0, 1, 2):
                    kj = _KI[c][sj]
                    if kj < 0:
                        continue
                    big = big.at[si, sj, :, r, c, :].set(wf[:, :, ki, kj].T)
    return big.reshape(9, ci, 4 * co)


def _pad1(x_cl):
    return jnp.pad(x_cl, ((0, 0), (1, 1), (1, 1), (0, 0)))


def kernel(imgs, w1, b1, w2, b2, codebook, wt1, bt1, wt2, bt2):
    n = imgs.shape[0]

    imp = _s2d(_pad1(imgs.transpose(0, 2, 3, 1)))
    h_cl = _conv_mm(_win4(imp, 192), _enc_weight(w1), b1[None, :], 48)

    hq = _s2d(_pad1(h_cl))
    ze_cl = _conv_mm(_win4(hq, 96), _enc_weight(w2), b2[None, :], 24)

    zflat = ze_cl.reshape(-1, _D)
    s = jnp.sum(codebook ** 2, axis=1)
    zq = _vq(zflat, s[None, :], codebook)
    zq_cl = zq.reshape(n, 96, 96, _D)

    xt1 = _win9(_pad1(zq_cl).astype(jnp.bfloat16), 96)
    d4 = _conv_mm(xt1, _dec_weight(wt1).astype(jnp.bfloat16),
                  jnp.tile(bt1, 4)[None, :], 24)
    d_cl = (d4.reshape(n, 96, 96, 2, 2, 16)
            .transpose(0, 1, 3, 2, 4, 5)
            .reshape(n, 192, 192, 16))

    xt2 = _win9(_pad1(d_cl).astype(jnp.bfloat16), 192)
    dec4 = _conv_mm(xt2, _dec_weight(wt2).astype(jnp.bfloat16),
                    jnp.tile(bt2, 4)[None, :], 16)
    decoded = (dec4.reshape(n, 192, 192, 2, 2, 3)
               .transpose(0, 1, 3, 2, 4, 5)
               .reshape(n, 384, 384, 3)
               .transpose(0, 3, 1, 2))

    z_e = ze_cl.transpose(0, 3, 1, 2)
    encoded = zq_cl.transpose(0, 3, 1, 2)
    return (z_e, encoded, decoded)


# speedup vs baseline: 1.5285x; 1.3482x over previous
import functools

import jax
import jax.numpy as jnp
from jax import lax
from jax.experimental import pallas as pl
from jax.experimental.pallas import tpu as pltpu, tpu_sc as plsc

_K, _D = 1024, 32
_VQB = 1024


def _idx_body(a_ref, s_ref, idx_ref):
    a = a_ref[...]
    d = jnp.abs(a - s_ref[...])
    m = jnp.min(d, axis=1, keepdims=True)
    iota = lax.broadcasted_iota(jnp.int32, (_VQB, _K), 1)
    idx_ref[...] = jnp.min(jnp.where(d == m, iota, _K), axis=1)[:, None]


def _vq_idx(a, s):
    n = a.shape[0]
    return pl.pallas_call(
        _idx_body,
        grid=(n // _VQB,),
        in_specs=[
            pl.BlockSpec((_VQB, 1), lambda i: (i, 0)),
            pl.BlockSpec((1, _K), lambda i: (0, 0)),
        ],
        out_specs=pl.BlockSpec((_VQB, 1), lambda i: (i, 0)),
        out_shape=jax.ShapeDtypeStruct((n, 1), jnp.int32),
    )(a, s)[:, 0]


def _sc_gather(table, idx):
    info = plsc.get_sparse_core_info()
    nw = info.num_cores * info.num_subcores
    b = idx.shape[0]
    d = table.shape[1]
    b_per_w = b // nw
    nchunk = 4
    bc = b_per_w // nchunk
    mesh = plsc.VectorSubcoreMesh(core_axis_name="c", subcore_axis_name="s")

    @functools.partial(
        pl.kernel, mesh=mesh,
        out_type=jax.ShapeDtypeStruct((b, d), jnp.float32),
        scratch_types=[
            pltpu.VMEM((bc,), jnp.int32),
            pltpu.VMEM((bc, d), jnp.float32),
            pltpu.SemaphoreType.DMA,
        ],
    )
    def k(table_hbm, idx_hbm, out_hbm, idx_v, rows_v, sem):
        wid = lax.axis_index("s") * info.num_cores + lax.axis_index("c")
        for ch in range(nchunk):
            base = wid * b_per_w + ch * bc
            pltpu.sync_copy(idx_hbm.at[pl.ds(base, bc)], idx_v)
            pltpu.async_copy(table_hbm.at[idx_v], rows_v, sem).wait()
            pltpu.sync_copy(rows_v, out_hbm.at[pl.ds(base, bc)])

    return k(table, idx)


def _conv(x, w, b, stride):
    y = lax.conv_general_dilated(x, w, window_strides=(stride, stride),
                                 padding=((1, 1), (1, 1)),
                                 dimension_numbers=('NCHW', 'OIHW', 'NCHW'))
    return y + b[None, :, None, None]


def _conv_t(x, w, b):
    wf = jnp.flip(w, axis=(2, 3)).transpose(1, 0, 2, 3)
    y = lax.conv_general_dilated(x, wf, window_strides=(1, 1),
                                 padding=((2, 2), (2, 2)),
                                 lhs_dilation=(2, 2),
                                 dimension_numbers=('NCHW', 'OIHW', 'NCHW'))
    return y + b[None, :, None, None]


def kernel(imgs, w1, b1, w2, b2, codebook, wt1, bt1, wt2, bt2):
    h = jax.nn.relu(_conv(imgs, w1, b1, 2))
    z_e = jax.nn.relu(_conv(h, w2, b2, 2))
    n, c, hh, ww = z_e.shape
    zflat = z_e.transpose(0, 2, 3, 1).reshape(-1, _D)
    a = jnp.sum(zflat ** 2, axis=1)
    s = jnp.sum(codebook ** 2, axis=1)
    idx = _vq_idx(a[:, None], s[None, :])
    cb_pad = jnp.pad(codebook, ((0, 0), (0, 128 - _D)))
    zq = _sc_gather(cb_pad, idx)[:, :_D]
    encoded = zq.reshape(n, hh, ww, _D).transpose(0, 3, 1, 2)
    d = jax.nn.relu(_conv_t(encoded, wt1, bt1))
    decoded = jax.nn.relu(_conv_t(d, wt2, bt2))
    return (z_e, encoded, decoded)


# speedup vs baseline: 4.1435x; 2.7108x over previous
import jax
import jax.numpy as jnp
from jax import lax
from jax.experimental import pallas as pl

_K, _D = 1024, 32
_BLK = 1024


def _vq_body(z_ref, s_ref, cb_ref, zq_ref):
    zb = z_ref[...]
    cb = cb_ref[...]
    a = jnp.sum(zb * zb, axis=1, keepdims=True)
    d = jnp.abs(a - s_ref[...])
    m = jnp.min(d, axis=1, keepdims=True)
    iota = lax.broadcasted_iota(jnp.int32, (_BLK, _K), 1)
    idx = jnp.min(jnp.where(d == m, iota, _K), axis=1)
    onehot = (lax.broadcasted_iota(jnp.int32, (_BLK, _K), 1)
              == idx[:, None]).astype(jnp.float32)
    zq_ref[...] = jnp.dot(onehot, cb, preferred_element_type=jnp.float32)


def _vq(zflat, s, codebook):
    n = zflat.shape[0]
    grid = n // _BLK
    return pl.pallas_call(
        _vq_body,
        grid=(grid,),
        in_specs=[
            pl.BlockSpec((_BLK, _D), lambda i: (i, 0)),
            pl.BlockSpec((1, _K), lambda i: (0, 0)),
            pl.BlockSpec((_K, _D), lambda i: (0, 0)),
        ],
        out_specs=pl.BlockSpec((_BLK, _D), lambda i: (i, 0)),
        out_shape=jax.ShapeDtypeStruct((n, _D), jnp.float32),
    )(zflat, s, codebook)


def _conv(x, w, b, stride):
    y = lax.conv_general_dilated(x, w, window_strides=(stride, stride),
                                 padding=((1, 1), (1, 1)),
                                 dimension_numbers=('NCHW', 'OIHW', 'NCHW'))
    return y + b[None, :, None, None]


def _conv_t(x, w, b):
    wf = jnp.flip(w, axis=(2, 3)).transpose(1, 0, 2, 3)
    y = lax.conv_general_dilated(x, wf, window_strides=(1, 1),
                                 padding=((2, 2), (2, 2)),
                                 lhs_dilation=(2, 2),
                                 dimension_numbers=('NCHW', 'OIHW', 'NCHW'))
    return y + b[None, :, None, None]


def kernel(imgs, w1, b1, w2, b2, codebook, wt1, bt1, wt2, bt2):
    h = jax.nn.relu(_conv(imgs, w1, b1, 2))
    z_e = jax.nn.relu(_conv(h, w2, b2, 2))
    n, c, hh, ww = z_e.shape
    zflat = z_e.transpose(0, 2, 3, 1).reshape(-1, _D)
    s = jnp.sum(codebook ** 2, axis=1)
    zq = _vq(zflat, s[None, :], codebook)
    encoded = zq.reshape(n, hh, ww, _D).transpose(0, 3, 1, 2)
    d = jax.nn.relu(_conv_t(encoded, wt1, bt1))
    decoded = jax.nn.relu(_conv_t(d, wt2, bt2))
    return (z_e, encoded, decoded)
